# baseline (speedup 1.0000x reference)
import jax
import jax.numpy as jnp
from jax import lax
from jax.experimental import pallas as pl
from jax.experimental.pallas import tpu as pltpu
from jax.experimental.pallas import tpu_sc as plsc

N_NODES = 10000
D_FEAT = 128
N_EDGES = 320000

_NC = 2
_NS = 16

_CHUNK = 32
_E_PAD = 327680
_SPAN_ROWS = 160
_NPH = 20
_PH_ROWS = _SPAN_ROWS // _NPH
_PH_CHUNKS = _PH_ROWS * 4

_HALF = 5056
_ACC_ROWS = 5064
_SINK = 5063

_XS_BIG = 632
_XS_LAST = 520
_AC_BIG = 320
_AC_LAST = 264


def _mp_body(x_hbm, ej_hbm, ei_hbm, zero_hbm, out_hbm,
             ejin, eiin, rows_v, st0, st1, x_sh, acc,
             gsem, ssem0, ssem1):
    c = lax.axis_index("c")
    s = lax.axis_index("s")

    @pl.when(s < _NS - 1)
    def _stage_main():
        pltpu.sync_copy(x_hbm.at[pl.ds(s * _XS_BIG, _XS_BIG)],
                        x_sh.at[pl.ds(s * _XS_BIG, _XS_BIG)])
        pltpu.sync_copy(zero_hbm.at[pl.ds(s * _AC_BIG, _AC_BIG)],
                        acc.at[pl.ds(s * _AC_BIG, _AC_BIG)])

    @pl.when(s == _NS - 1)
    def _stage_last():
        pltpu.sync_copy(x_hbm.at[pl.ds(15 * _XS_BIG, _XS_LAST)],
                        x_sh.at[pl.ds(15 * _XS_BIG, _XS_LAST)])
        pltpu.sync_copy(zero_hbm.at[pl.ds(15 * _AC_BIG, _AC_LAST)],
                        acc.at[pl.ds(15 * _AC_BIG, _AC_LAST)])

    plsc.subcore_barrier()

    dst_lo = c * _HALF
    span0 = s * _SPAN_ROWS

    def build_stage(row, col, stage):
        for g in range(2):
            eig = eiin[row, pl.ds(col + g * 16, 16)]
            loc = eig - dst_lo
            m = (loc >= 0) & (loc < _HALF)
            stage[pl.ds(g * 16, 16)] = jnp.where(m, loc, _SINK)

    def gather_desc(row, col, p):
        return pltpu.make_async_copy(
            x_sh.at[ejin.at[row, pl.ds(col, 32)]], rows_v.at[p], gsem)

    def phase_body(ph, carry):
        pltpu.sync_copy(ej_hbm.at[pl.ds(span0 + ph * _PH_ROWS, _PH_ROWS)],
                        ejin)
        pltpu.sync_copy(ei_hbm.at[pl.ds(span0 + ph * _PH_ROWS, _PH_ROWS)],
                        eiin)

        def body(i, ic):
            k0 = 2 * i
            row0 = k0 // 4
            col0 = (k0 % 4) * 32
            k1 = k0 + 1
            row1 = k1 // 4
            col1 = (k1 % 4) * 32

            @pl.when(i >= 1)
            def _drain_even():
                pltpu.make_async_copy(rows_v.at[0], acc.at[st0],
                                      ssem0).wait()

            build_stage(row0, col0, st0)
            gather_desc(row0, col0, 0).start()

            @pl.when(i >= 1)
            def _drain_odd():
                pltpu.make_async_copy(rows_v.at[1], acc.at[st1],
                                      ssem1).wait()

            build_stage(row1, col1, st1)
            gather_desc(row0, col0, 0).wait()
            pltpu.async_copy(rows_v.at[0], acc.at[st0], ssem0, add=True)
            gather_desc(row1, col1, 1).start()
            gather_desc(row1, col1, 1).wait()
            pltpu.async_copy(rows_v.at[1], acc.at[st1], ssem1, add=True)
            return ic

        lax.fori_loop(0, _PH_CHUNKS // 2, body, 0)
        pltpu.make_async_copy(rows_v.at[0], acc.at[st0], ssem0).wait()
        pltpu.make_async_copy(rows_v.at[1], acc.at[st1], ssem1).wait()
        return carry

    lax.fori_loop(0, _NPH, phase_body, 0)
    plsc.subcore_barrier()

    @pl.when(s < _NS - 1)
    def _wb_main():
        pltpu.sync_copy(acc.at[pl.ds(s * _AC_BIG, _AC_BIG)],
                        out_hbm.at[c, pl.ds(s * _AC_BIG, _AC_BIG)])

    @pl.when(s == _NS - 1)
    def _wb_last():
        pltpu.sync_copy(acc.at[pl.ds(15 * _AC_BIG, _AC_LAST)],
                        out_hbm.at[c, pl.ds(15 * _AC_BIG, _AC_LAST)])


def kernel(x, edge_index):
    ej = edge_index[0].astype(jnp.int32)
    ei = edge_index[1].astype(jnp.int32)
    pad = _E_PAD - N_EDGES
    ej = jnp.concatenate([ej, jnp.zeros((pad,), jnp.int32)])
    ei = jnp.concatenate([ei, jnp.full((pad,), N_NODES, jnp.int32)])
    ej2 = ej.reshape(_E_PAD // 128, 128)
    ei2 = ei.reshape(_E_PAD // 128, 128)
    zeros = jnp.zeros((_ACC_ROWS, D_FEAT), jnp.float32)

    mesh = plsc.VectorSubcoreMesh(core_axis_name="c", subcore_axis_name="s")
    partials = pl.kernel(
        _mp_body,
        mesh=mesh,
        out_type=jax.ShapeDtypeStruct((_NC, _ACC_ROWS, D_FEAT), jnp.float32),
        scratch_types=[
            pltpu.VMEM((_PH_ROWS, 128), jnp.int32),
            pltpu.VMEM((_PH_ROWS, 128), jnp.int32),
            pltpu.VMEM((2, _CHUNK, D_FEAT), jnp.float32),
            pltpu.VMEM((_CHUNK,), jnp.int32),
            pltpu.VMEM((_CHUNK,), jnp.int32),
            pltpu.VMEM_SHARED((N_NODES, D_FEAT), jnp.float32),
            pltpu.VMEM_SHARED((_ACC_ROWS, D_FEAT), jnp.float32),
            pltpu.SemaphoreType.DMA,
            pltpu.SemaphoreType.DMA,
            pltpu.SemaphoreType.DMA,
        ],
    )(x, ej2, ei2, zeros)

    out = jnp.concatenate(
        [partials[0, :_HALF, :], partials[1, :N_NODES - _HALF, :]])
    return out


# speedup vs baseline: 1.0237x; 1.0237x over previous
import jax
import jax.numpy as jnp
from jax import lax
from jax.experimental import pallas as pl
from jax.experimental.pallas import tpu as pltpu
from jax.experimental.pallas import tpu_sc as plsc

N_NODES = 10000
D_FEAT = 128
N_EDGES = 320000

_NC = 2
_NS = 16

_CHUNK = 32
_E_PAD = 327680
_SPAN_ROWS = 160
_NPH = 20
_PH_ROWS = _SPAN_ROWS // _NPH
_PH_CHUNKS = _PH_ROWS * 4

_HALF = 5056
_ACC_ROWS = 5064
_SINK = 5063

_XS_BIG = 632
_XS_LAST = 520
_AC_BIG = 320
_AC_LAST = 264
_WB_LAST = 256
_WB_HI = 144


def _mp_body(x_hbm, ej_hbm, ei_hbm, zero_hbm, out_hbm,
             ejin, eiin, rows_v, st0, st1, x_sh, acc,
             gsem, ssem0, ssem1):
    c = lax.axis_index("c")
    s = lax.axis_index("s")

    @pl.when(s < _NS - 1)
    def _stage_main():
        pltpu.sync_copy(x_hbm.at[pl.ds(s * _XS_BIG, _XS_BIG)],
                        x_sh.at[pl.ds(s * _XS_BIG, _XS_BIG)])
        pltpu.sync_copy(zero_hbm.at[pl.ds(s * _AC_BIG, _AC_BIG)],
                        acc.at[pl.ds(s * _AC_BIG, _AC_BIG)])

    @pl.when(s == _NS - 1)
    def _stage_last():
        pltpu.sync_copy(x_hbm.at[pl.ds(15 * _XS_BIG, _XS_LAST)],
                        x_sh.at[pl.ds(15 * _XS_BIG, _XS_LAST)])
        pltpu.sync_copy(zero_hbm.at[pl.ds(15 * _AC_BIG, _AC_LAST)],
                        acc.at[pl.ds(15 * _AC_BIG, _AC_LAST)])

    plsc.subcore_barrier()

    dst_lo = c * _HALF
    span0 = s * _SPAN_ROWS

    def build_stage(row, col, stage):
        for g in range(2):
            eig = eiin[row, pl.ds(col + g * 16, 16)]
            loc = eig - dst_lo
            m = (loc >= 0) & (loc < _HALF)
            stage[pl.ds(g * 16, 16)] = jnp.where(m, loc, _SINK)

    def gather_desc(row, col, p):
        return pltpu.make_async_copy(
            x_sh.at[ejin.at[row, pl.ds(col, 32)]], rows_v.at[p], gsem)

    def phase_body(ph, carry):
        pltpu.sync_copy(ej_hbm.at[pl.ds(span0 + ph * _PH_ROWS, _PH_ROWS)],
                        ejin)
        pltpu.sync_copy(ei_hbm.at[pl.ds(span0 + ph * _PH_ROWS, _PH_ROWS)],
                        eiin)

        def body(i, ic):
            k0 = 2 * i
            row0 = k0 // 4
            col0 = (k0 % 4) * 32
            k1 = k0 + 1
            row1 = k1 // 4
            col1 = (k1 % 4) * 32

            @pl.when(i >= 1)
            def _drain_even():
                pltpu.make_async_copy(rows_v.at[0], acc.at[st0],
                                      ssem0).wait()

            build_stage(row0, col0, st0)
            gather_desc(row0, col0, 0).start()

            @pl.when(i >= 1)
            def _drain_odd():
                pltpu.make_async_copy(rows_v.at[1], acc.at[st1],
                                      ssem1).wait()

            build_stage(row1, col1, st1)
            gather_desc(row0, col0, 0).wait()
            pltpu.async_copy(rows_v.at[0], acc.at[st0], ssem0, add=True)
            gather_desc(row1, col1, 1).start()
            gather_desc(row1, col1, 1).wait()
            pltpu.async_copy(rows_v.at[1], acc.at[st1], ssem1, add=True)
            return ic

        lax.fori_loop(0, _PH_CHUNKS // 2, body, 0)
        pltpu.make_async_copy(rows_v.at[0], acc.at[st0], ssem0).wait()
        pltpu.make_async_copy(rows_v.at[1], acc.at[st1], ssem1).wait()
        return carry

    lax.fori_loop(0, _NPH, phase_body, 0)
    plsc.subcore_barrier()

    @pl.when(s < _NS - 1)
    def _wb_main():
        pltpu.sync_copy(acc.at[pl.ds(s * _AC_BIG, _AC_BIG)],
                        out_hbm.at[pl.ds(c * _HALF + s * _AC_BIG, _AC_BIG)])

    @pl.when((s == _NS - 1) & (c == 0))
    def _wb_last_lo():
        pltpu.sync_copy(acc.at[pl.ds(15 * _AC_BIG, _WB_LAST)],
                        out_hbm.at[pl.ds(15 * _AC_BIG, _WB_LAST)])

    @pl.when((s == _NS - 1) & (c == 1))
    def _wb_last_hi():
        pltpu.sync_copy(acc.at[pl.ds(15 * _AC_BIG, _WB_HI)],
                        out_hbm.at[pl.ds(_HALF + 15 * _AC_BIG, _WB_HI)])


def kernel(x, edge_index):
    ej = edge_index[0].astype(jnp.int32)
    ei = edge_index[1].astype(jnp.int32)
    pad = _E_PAD - N_EDGES
    ej = jnp.concatenate([ej, jnp.zeros((pad,), jnp.int32)])
    ei = jnp.concatenate([ei, jnp.full((pad,), N_NODES, jnp.int32)])
    ej2 = ej.reshape(_E_PAD // 128, 128)
    ei2 = ei.reshape(_E_PAD // 128, 128)
    zeros = jnp.zeros((_ACC_ROWS, D_FEAT), jnp.float32)

    mesh = plsc.VectorSubcoreMesh(core_axis_name="c", subcore_axis_name="s")
    partials = pl.kernel(
        _mp_body,
        mesh=mesh,
        out_type=jax.ShapeDtypeStruct((N_NODES, D_FEAT), jnp.float32),
        scratch_types=[
            pltpu.VMEM((_PH_ROWS, 128), jnp.int32),
            pltpu.VMEM((_PH_ROWS, 128), jnp.int32),
            pltpu.VMEM((2, _CHUNK, D_FEAT), jnp.float32),
            pltpu.VMEM((_CHUNK,), jnp.int32),
            pltpu.VMEM((_CHUNK,), jnp.int32),
            pltpu.VMEM_SHARED((N_NODES, D_FEAT), jnp.float32),
            pltpu.VMEM_SHARED((_ACC_ROWS, D_FEAT), jnp.float32),
            pltpu.SemaphoreType.DMA,
            pltpu.SemaphoreType.DMA,
            pltpu.SemaphoreType.DMA,
        ],
    )(x, ej2, ei2, zeros)

    return partials


# speedup vs baseline: 1.0674x; 1.0427x over previous
import jax
import jax.numpy as jnp
from jax import lax
from jax.experimental import pallas as pl
from jax.experimental.pallas import tpu as pltpu
from jax.experimental.pallas import tpu_sc as plsc

N_NODES = 10000
D_FEAT = 128
N_EDGES = 320000

_NC = 2
_NS = 16

_CHUNK = 16
_E_PAD = 327680
_SPAN_ROWS = 160
_NPH = 20
_PH_ROWS = _SPAN_ROWS // _NPH
_PH_CHUNKS = _PH_ROWS * 8

_HALF = 5056
_ACC_ROWS = 5064
_SINK = 5063

_XS_BIG = 632
_XS_LAST = 520
_AC_BIG = 320
_AC_LAST = 264
_WB_LAST = 256
_WB_HI = 144


def _mp_body(x_hbm, ej_hbm, ei_hbm, zero_hbm, out_hbm,
             ejin, eiin, rows_v, stg, x_sh, acc,
             gsem0, gsem1, ssem0, ssem1, ssem2, ssem3):
    c = lax.axis_index("c")
    s = lax.axis_index("s")

    @pl.when(s < _NS - 1)
    def _stage_main():
        pltpu.sync_copy(x_hbm.at[pl.ds(s * _XS_BIG, _XS_BIG)],
                        x_sh.at[pl.ds(s * _XS_BIG, _XS_BIG)])
        pltpu.sync_copy(zero_hbm.at[pl.ds(s * _AC_BIG, _AC_BIG)],
                        acc.at[pl.ds(s * _AC_BIG, _AC_BIG)])

    @pl.when(s == _NS - 1)
    def _stage_last():
        pltpu.sync_copy(x_hbm.at[pl.ds(15 * _XS_BIG, _XS_LAST)],
                        x_sh.at[pl.ds(15 * _XS_BIG, _XS_LAST)])
        pltpu.sync_copy(zero_hbm.at[pl.ds(15 * _AC_BIG, _AC_LAST)],
                        acc.at[pl.ds(15 * _AC_BIG, _AC_LAST)])

    plsc.subcore_barrier()

    dst_lo = c * _HALF
    span0 = s * _SPAN_ROWS

    def build_stage(k, j):
        row = k // 8
        col = (k % 8) * 16
        eig = eiin[row, pl.ds(col, 16)]
        loc = eig - dst_lo
        m = (loc >= 0) & (loc < _HALF)
        stg[pl.ds(16 * j, 16)] = jnp.where(m, loc, _SINK)

    def gather_desc(k, b, gsem):
        row = k // 8
        col = (k % 8) * 16
        return pltpu.make_async_copy(
            x_sh.at[ejin.at[row, pl.ds(col, 16)]], rows_v.at[b], gsem)

    def phase_body(ph, carry):
        pltpu.sync_copy(ej_hbm.at[pl.ds(span0 + ph * _PH_ROWS, _PH_ROWS)],
                        ejin)
        pltpu.sync_copy(ei_hbm.at[pl.ds(span0 + ph * _PH_ROWS, _PH_ROWS)],
                        eiin)
        gather_desc(0, 0, gsem0).start()
        gather_desc(1, 1, gsem1).start()

        def quad(i, ic):
            for j, (gsem, ssm, ssn) in enumerate((
                    (gsem0, ssem0, ssem2),
                    (gsem1, ssem1, ssem3),
                    (gsem0, ssem2, ssem0),
                    (gsem1, ssem3, ssem1))):
                k = 4 * i + j
                b = j
                bn = (j + 2) % 4
                gather_desc(k, b, gsem).wait()
                build_stage(k, j)
                pltpu.async_copy(rows_v.at[b],
                                 acc.at[stg.at[pl.ds(16 * j, 16)]],
                                 ssm, add=True)

                @pl.when(k >= 2)
                def _drain_old():
                    pltpu.make_async_copy(
                        rows_v.at[bn],
                        acc.at[stg.at[pl.ds(16 * bn, 16)]], ssn).wait()

                @pl.when(k + 2 < _PH_CHUNKS)
                def _prefetch():
                    gather_desc(k + 2, bn, gsem).start()

            return ic

        lax.fori_loop(0, _PH_CHUNKS // 4, quad, 0)
        pltpu.make_async_copy(rows_v.at[2],
                              acc.at[stg.at[pl.ds(32, 16)]], ssem2).wait()
        pltpu.make_async_copy(rows_v.at[3],
                              acc.at[stg.at[pl.ds(48, 16)]], ssem3).wait()
        return carry

    lax.fori_loop(0, _NPH, phase_body, 0)
    plsc.subcore_barrier()

    @pl.when(s < _NS - 1)
    def _wb_main():
        pltpu.sync_copy(acc.at[pl.ds(s * _AC_BIG, _AC_BIG)],
                        out_hbm.at[pl.ds(c * _HALF + s * _AC_BIG, _AC_BIG)])

    @pl.when((s == _NS - 1) & (c == 0))
    def _wb_last_lo():
        pltpu.sync_copy(acc.at[pl.ds(15 * _AC_BIG, _WB_LAST)],
                        out_hbm.at[pl.ds(15 * _AC_BIG, _WB_LAST)])

    @pl.when((s == _NS - 1) & (c == 1))
    def _wb_last_hi():
        pltpu.sync_copy(acc.at[pl.ds(15 * _AC_BIG, _WB_HI)],
                        out_hbm.at[pl.ds(_HALF + 15 * _AC_BIG, _WB_HI)])


def kernel(x, edge_index):
    ej = edge_index[0].astype(jnp.int32)
    ei = edge_index[1].astype(jnp.int32)
    pad = _E_PAD - N_EDGES
    ej = jnp.concatenate([ej, jnp.zeros((pad,), jnp.int32)])
    ei = jnp.concatenate([ei, jnp.full((pad,), N_NODES, jnp.int32)])
    ej2 = ej.reshape(_E_PAD // 128, 128)
    ei2 = ei.reshape(_E_PAD // 128, 128)
    zeros = jnp.zeros((_ACC_ROWS, D_FEAT), jnp.float32)

    mesh = plsc.VectorSubcoreMesh(core_axis_name="c", subcore_axis_name="s")
    partials = pl.kernel(
        _mp_body,
        mesh=mesh,
        out_type=jax.ShapeDtypeStruct((N_NODES, D_FEAT), jnp.float32),
        scratch_types=[
            pltpu.VMEM((_PH_ROWS, 128), jnp.int32),
            pltpu.VMEM((_PH_ROWS, 128), jnp.int32),
            pltpu.VMEM((4, _CHUNK, D_FEAT), jnp.float32),
            pltpu.VMEM((4 * _CHUNK,), jnp.int32),
            pltpu.VMEM_SHARED((N_NODES, D_FEAT), jnp.float32),
            pltpu.VMEM_SHARED((_ACC_ROWS, D_FEAT), jnp.float32),
            pltpu.SemaphoreType.DMA,
            pltpu.SemaphoreType.DMA,
            pltpu.SemaphoreType.DMA,
            pltpu.SemaphoreType.DMA,
            pltpu.SemaphoreType.DMA,
            pltpu.SemaphoreType.DMA,
        ],
    )(x, ej2, ei2, zeros)

    return partials
